# baseline (speedup 1.0000x reference)
import functools

import jax
import jax.numpy as jnp
from jax import lax
from jax.experimental import pallas as pl
from jax.experimental.pallas import tpu as pltpu
from jax.experimental.pallas import tpu_sc as plsc

B, S, D_IN, HID, VOCAB, MAX_LEN = 16, 8, 256, 512, 128, 512
R = B * S
FLAT = S * MAX_LEN


K_SUB = 8
G = MAX_LEN // K_SUB
W = K_SUB * R


def _rnn_body(xt_ref, wint_ref, wht_ref, woutt_ref,
              tok_ref, lp_ref, ent_ref, dest_ref, lbuf_ref):
    h0 = jnp.tanh(jnp.dot(wint_ref[...], xt_ref[...],
                          preferred_element_type=jnp.float32))
    wht = wht_ref[...]
    woutt = woutt_ref[...]
    vocab_iota = lax.broadcasted_iota(jnp.int32, (VOCAB, W), 0)

    def group(g, h):
        for k in range(K_SUB):
            h = jnp.tanh(jnp.dot(wht, h, preferred_element_type=jnp.float32))
            lbuf_ref[:, k * R:(k + 1) * R] = jnp.dot(
                woutt, h, preferred_element_type=jnp.float32)
        logits = lbuf_ref[...]
        m = jnp.max(logits, axis=0, keepdims=True)
        shifted = logits - m
        e = jnp.exp(shifted)
        ssum = jnp.sum(e, axis=0, keepdims=True)
        tok = jnp.min(jnp.where(logits == m, vocab_iota, VOCAB),
                      axis=0, keepdims=True)
        logs = jnp.log(ssum)
        lp = -logs
        ent = logs - jnp.sum(e * shifted, axis=0, keepdims=True) / ssum
        tok_ref[pl.ds(g, 1), :] = tok
        lp_ref[pl.ds(g, 1), :] = lp
        ent_ref[pl.ds(g, 1), :] = ent
        return h

    lax.fori_loop(0, G, group, h0)

    toks = tok_ref[...]
    c_iota = lax.broadcasted_iota(jnp.int32, (G, W), 1)
    g_iota = lax.broadcasted_iota(jnp.int32, (G, W), 0)
    pos2 = g_iota * K_SUB + c_iota // R
    colmin = jnp.min(jnp.where(toks == 0, pos2, MAX_LEN),
                     axis=0, keepdims=True)
    fz = colmin[:, 0:R]
    for k in range(1, K_SUB):
        fz = jnp.minimum(fz, colmin[:, k * R:(k + 1) * R])
    length = jnp.minimum(fz + 1, MAX_LEN)
    p = length - 1
    p_f = p.astype(jnp.float32)

    r0 = lax.broadcasted_iota(jnp.int32, (R, R), 0)
    r1 = lax.broadcasted_iota(jnp.int32, (R, R), 1)
    same_b = (r0 // S) == (r1 // S)
    mt_strict = jnp.where(same_b & (r0 < r1), 1.0, 0.0).astype(jnp.float32)
    mt_block = jnp.where(same_b, 1.0, 0.0).astype(jnp.float32)
    fo = jnp.dot(p_f, mt_strict, preferred_element_type=jnp.float32,
                 precision=lax.Precision.HIGHEST)
    tf = jnp.dot(p_f, mt_block, preferred_element_type=jnp.float32,
                 precision=lax.Precision.HIGHEST)
    s_idx = lax.broadcasted_iota(jnp.int32, (1, R), 1) % S
    lo = s_idx.astype(jnp.float32) * MAX_LEN - fo

    def tile8(v):
        return jnp.concatenate([v] * K_SUB, axis=1)

    p_t = tile8(p)
    pos2_f = pos2.astype(jnp.float32)
    dest_f = jnp.where(pos2 < p_t,
                       tile8(fo) + pos2_f,
                       tile8(tf + lo) + pos2_f - tile8(p_f))
    dest_ref[...] = dest_f.astype(jnp.int32)


_rnn_call = pl.pallas_call(
    _rnn_body,
    out_shape=[
        jax.ShapeDtypeStruct((G, W), jnp.int32),
        jax.ShapeDtypeStruct((G, W), jnp.float32),
        jax.ShapeDtypeStruct((G, W), jnp.float32),
        jax.ShapeDtypeStruct((G, W), jnp.int32),
    ],
    scratch_shapes=[pltpu.VMEM((VOCAB, W), jnp.float32)],
)


_NC, _NS = 2, 16
_NW = _NC * _NS


def _sc_scatter_body(vals_f_hbm, toks_hbm, dest_hbm, out_f_hbm, out_i_hbm,
                     val_v, tok_v, orow_f, orow_i, idx_v):
    wid = lax.axis_index("s") * _NC + lax.axis_index("c")
    b = lax.rem(wid, B)
    pltpu.sync_copy(vals_f_hbm.at[wid], val_v)
    pltpu.sync_copy(dest_hbm.at[b], idx_v)

    def body_f(i, carry):
        sl = pl.ds(i * 16, 16)
        plsc.store_scatter(orow_f, [idx_v[sl]], val_v[sl])
        return carry

    lax.fori_loop(0, FLAT // 16, body_f, 0)
    pltpu.sync_copy(orow_f, out_f_hbm.at[wid])

    @pl.when(wid < B)
    def _():
        pltpu.sync_copy(toks_hbm.at[wid], tok_v)

        def body_i(i, carry):
            sl = pl.ds(i * 16, 16)
            plsc.store_scatter(orow_i, [idx_v[sl]], tok_v[sl])
            return carry

        lax.fori_loop(0, FLAT // 16, body_i, 0)
        pltpu.sync_copy(orow_i, out_i_hbm.at[wid])


@functools.cache
def _make_sc_scatter():
    return pl.kernel(
        _sc_scatter_body,
        mesh=plsc.VectorSubcoreMesh(
            core_axis_name="c", subcore_axis_name="s",
            num_cores=_NC, num_subcores=_NS),
        out_type=[
            jax.ShapeDtypeStruct((2 * B, FLAT), jnp.float32),
            jax.ShapeDtypeStruct((B, FLAT), jnp.int32),
        ],
        scratch_types=[
            pltpu.VMEM((FLAT,), jnp.float32),
            pltpu.VMEM((FLAT,), jnp.int32),
            pltpu.VMEM((FLAT,), jnp.float32),
            pltpu.VMEM((FLAT,), jnp.int32),
            pltpu.VMEM((FLAT,), jnp.int32),
        ],
        compiler_params=pltpu.CompilerParams(needs_layout_passes=False),
    )


def _to_rows(a):
    return a.reshape(G, K_SUB, R).transpose(2, 0, 1).reshape(B, FLAT)


@jax.jit
def kernel(input, W_in, W_h, W_out):
    xt = input.reshape(R, D_IN).T
    toks_t, lp_t, ent_t, dest_t = _rnn_call(xt, W_in.T, W_h.T, W_out.T)
    vals_f = jnp.concatenate([_to_rows(lp_t), _to_rows(ent_t)], axis=0)
    out_f, out_i = _make_sc_scatter()(vals_f, _to_rows(toks_t),
                                      _to_rows(dest_t))
    return (out_i, out_f[:B], out_f[B:])


# speedup vs baseline: 1.1031x; 1.1031x over previous
import functools

import jax
import jax.numpy as jnp
from jax import lax
from jax.experimental import pallas as pl
from jax.experimental.pallas import tpu as pltpu
from jax.experimental.pallas import tpu_sc as plsc

B, S, D_IN, HID, VOCAB, MAX_LEN = 16, 8, 256, 512, 128, 512
R = B * S
FLAT = S * MAX_LEN


def _rnn_body(xt_ref, wint_ref, wht_ref, woutt_ref,
              tok_out_ref, lp_out_ref, ent_out_ref, dest_out_ref,
              tok_ref, lp_ref, ent_ref):
    h0 = jnp.tanh(jnp.dot(wint_ref[...], xt_ref[...],
                          preferred_element_type=jnp.float32))
    wht = wht_ref[...]
    woutt = woutt_ref[...]
    vocab_iota = lax.broadcasted_iota(jnp.int32, (VOCAB, R), 0)

    def step(t, h):
        h = jnp.tanh(jnp.dot(wht, h, preferred_element_type=jnp.float32))
        logits = jnp.dot(woutt, h, preferred_element_type=jnp.float32)
        m = jnp.max(logits, axis=0, keepdims=True)
        shifted = logits - m
        e = jnp.exp(shifted)
        ssum = jnp.sum(e, axis=0, keepdims=True)
        tok = jnp.min(jnp.where(logits == m, vocab_iota, VOCAB),
                      axis=0, keepdims=True)
        logs = jnp.log(ssum)
        lp = -logs
        ent = logs - jnp.sum(e * shifted, axis=0, keepdims=True) / ssum
        tok_ref[pl.ds(t, 1), :] = tok
        lp_ref[pl.ds(t, 1), :] = lp
        ent_ref[pl.ds(t, 1), :] = ent
        return h

    lax.fori_loop(0, MAX_LEN, step, h0, unroll=8)

    toks = tok_ref[...]
    pos = lax.broadcasted_iota(jnp.int32, (MAX_LEN, R), 0)
    fz = jnp.min(jnp.where(toks == 0, pos, MAX_LEN), axis=0, keepdims=True)
    length = jnp.minimum(fz + 1, MAX_LEN)
    p = length - 1
    p_f = p.astype(jnp.float32)

    r0 = lax.broadcasted_iota(jnp.int32, (R, R), 0)
    r1 = lax.broadcasted_iota(jnp.int32, (R, R), 1)
    same_b = (r0 // S) == (r1 // S)
    mt_strict = jnp.where(same_b & (r0 < r1), 1.0, 0.0).astype(jnp.float32)
    mt_block = jnp.where(same_b, 1.0, 0.0).astype(jnp.float32)
    fo = jnp.dot(p_f, mt_strict, preferred_element_type=jnp.float32,
                 precision=lax.Precision.HIGHEST)
    tf = jnp.dot(p_f, mt_block, preferred_element_type=jnp.float32,
                 precision=lax.Precision.HIGHEST)
    s_idx = lax.broadcasted_iota(jnp.int32, (1, R), 1) % S
    lo = s_idx.astype(jnp.float32) * MAX_LEN - fo
    pos_f = pos.astype(jnp.float32)
    dest_f = jnp.where(pos < p, fo + pos_f, tf + lo + pos_f - p_f)

    tok_out_ref[...] = toks.T
    lp_out_ref[...] = lp_ref[...].T
    ent_out_ref[...] = ent_ref[...].T
    dest_out_ref[...] = dest_f.astype(jnp.int32).T


_rnn_call = pl.pallas_call(
    _rnn_body,
    out_shape=[
        jax.ShapeDtypeStruct((R, MAX_LEN), jnp.int32),
        jax.ShapeDtypeStruct((R, MAX_LEN), jnp.float32),
        jax.ShapeDtypeStruct((R, MAX_LEN), jnp.float32),
        jax.ShapeDtypeStruct((R, MAX_LEN), jnp.int32),
    ],
    scratch_shapes=[
        pltpu.VMEM((MAX_LEN, R), jnp.int32),
        pltpu.VMEM((MAX_LEN, R), jnp.float32),
        pltpu.VMEM((MAX_LEN, R), jnp.float32),
    ],
)


_NC, _NS = 2, 16
_NW = _NC * _NS


def _sc_scatter_body(vals_f_hbm, toks_hbm, dest_hbm, out_f_hbm, out_i_hbm,
                     val_v, tok_v, orow_f, orow_i, idx_v):
    wid = lax.axis_index("s") * _NC + lax.axis_index("c")
    b = lax.rem(wid, B)
    pltpu.sync_copy(vals_f_hbm.at[wid], val_v)
    pltpu.sync_copy(dest_hbm.at[b], idx_v)

    def body_f(i, carry):
        sl = pl.ds(i * 16, 16)
        plsc.store_scatter(orow_f, [idx_v[sl]], val_v[sl])
        return carry

    lax.fori_loop(0, FLAT // 16, body_f, 0)
    pltpu.sync_copy(orow_f, out_f_hbm.at[wid])

    @pl.when(wid < B)
    def _():
        pltpu.sync_copy(toks_hbm.at[wid], tok_v)

        def body_i(i, carry):
            sl = pl.ds(i * 16, 16)
            plsc.store_scatter(orow_i, [idx_v[sl]], tok_v[sl])
            return carry

        lax.fori_loop(0, FLAT // 16, body_i, 0)
        pltpu.sync_copy(orow_i, out_i_hbm.at[wid])


@functools.cache
def _make_sc_scatter():
    return pl.kernel(
        _sc_scatter_body,
        mesh=plsc.VectorSubcoreMesh(
            core_axis_name="c", subcore_axis_name="s",
            num_cores=_NC, num_subcores=_NS),
        out_type=[
            jax.ShapeDtypeStruct((2 * B, FLAT), jnp.float32),
            jax.ShapeDtypeStruct((B, FLAT), jnp.int32),
        ],
        scratch_types=[
            pltpu.VMEM((FLAT,), jnp.float32),
            pltpu.VMEM((FLAT,), jnp.int32),
            pltpu.VMEM((FLAT,), jnp.float32),
            pltpu.VMEM((FLAT,), jnp.int32),
            pltpu.VMEM((FLAT,), jnp.int32),
        ],
        compiler_params=pltpu.CompilerParams(needs_layout_passes=False),
    )


@jax.jit
def kernel(input, W_in, W_h, W_out):
    xt = input.reshape(R, D_IN).T
    toks_r, lp_r, ent_r, dest_r = _rnn_call(xt, W_in.T, W_h.T, W_out.T)
    vals_f = jnp.concatenate(
        [lp_r.reshape(B, FLAT), ent_r.reshape(B, FLAT)], axis=0)
    out_f, out_i = _make_sc_scatter()(vals_f, toks_r.reshape(B, FLAT),
                                      dest_r.reshape(B, FLAT))
    return (out_i, out_f[:B], out_f[B:])


# speedup vs baseline: 1.1220x; 1.0171x over previous
import functools

import jax
import jax.numpy as jnp
from jax import lax
from jax.experimental import pallas as pl
from jax.experimental.pallas import tpu as pltpu
from jax.experimental.pallas import tpu_sc as plsc

B, S, D_IN, HID, VOCAB, MAX_LEN = 16, 8, 256, 512, 128, 512
R = B * S
FLAT = S * MAX_LEN


def _rnn_body(xt_ref, wint_ref, wht_ref, woutt_ref,
              tok_out_ref, lp_out_ref, ent_out_ref, dest_out_ref,
              tok_ref, lp_ref, ent_ref):
    h0 = jnp.tanh(jnp.dot(wint_ref[...], xt_ref[...],
                          preferred_element_type=jnp.float32))
    wht = wht_ref[...]
    woutt = woutt_ref[...]
    vocab_iota = lax.broadcasted_iota(jnp.int32, (VOCAB, R), 0)

    def step(t, h):
        h = jnp.tanh(jnp.dot(wht, h, preferred_element_type=jnp.float32))
        logits = jnp.dot(woutt, h, preferred_element_type=jnp.float32)
        m = jnp.max(logits, axis=0, keepdims=True)
        shifted = logits - m
        e = jnp.exp(shifted)
        ssum = jnp.sum(e, axis=0, keepdims=True)
        tok = jnp.min(jnp.where(logits == m, vocab_iota, VOCAB),
                      axis=0, keepdims=True)
        logs = jnp.log(ssum)
        lp = -logs
        ent = logs - jnp.sum(e * shifted, axis=0, keepdims=True) / ssum
        tok_ref[pl.ds(t, 1), :] = tok
        lp_ref[pl.ds(t, 1), :] = lp
        ent_ref[pl.ds(t, 1), :] = ent
        return h

    lax.fori_loop(0, MAX_LEN, step, h0, unroll=16)

    toks = tok_ref[...]
    pos = lax.broadcasted_iota(jnp.int32, (MAX_LEN, R), 0)
    fz = jnp.min(jnp.where(toks == 0, pos, MAX_LEN), axis=0, keepdims=True)
    length = jnp.minimum(fz + 1, MAX_LEN)
    p = length - 1
    p_f = p.astype(jnp.float32)

    r0 = lax.broadcasted_iota(jnp.int32, (R, R), 0)
    r1 = lax.broadcasted_iota(jnp.int32, (R, R), 1)
    same_b = (r0 // S) == (r1 // S)
    mt_strict = jnp.where(same_b & (r0 < r1), 1.0, 0.0).astype(jnp.float32)
    mt_block = jnp.where(same_b, 1.0, 0.0).astype(jnp.float32)
    fo = jnp.dot(p_f, mt_strict, preferred_element_type=jnp.float32,
                 precision=lax.Precision.HIGHEST)
    tf = jnp.dot(p_f, mt_block, preferred_element_type=jnp.float32,
                 precision=lax.Precision.HIGHEST)
    s_idx = lax.broadcasted_iota(jnp.int32, (1, R), 1) % S
    lo = s_idx.astype(jnp.float32) * MAX_LEN - fo
    pos_f = pos.astype(jnp.float32)
    dest_f = jnp.where(pos < p, fo + pos_f, tf + lo + pos_f - p_f)

    tok_out_ref[...] = toks.T
    lp_out_ref[...] = lp_ref[...].T
    ent_out_ref[...] = ent_ref[...].T
    dest_out_ref[...] = dest_f.astype(jnp.int32).T


_rnn_call = pl.pallas_call(
    _rnn_body,
    out_shape=[
        jax.ShapeDtypeStruct((R, MAX_LEN), jnp.int32),
        jax.ShapeDtypeStruct((R, MAX_LEN), jnp.float32),
        jax.ShapeDtypeStruct((R, MAX_LEN), jnp.float32),
        jax.ShapeDtypeStruct((R, MAX_LEN), jnp.int32),
    ],
    scratch_shapes=[
        pltpu.VMEM((MAX_LEN, R), jnp.int32),
        pltpu.VMEM((MAX_LEN, R), jnp.float32),
        pltpu.VMEM((MAX_LEN, R), jnp.float32),
    ],
)


_NC, _NS = 2, 16
_NW = _NC * _NS


def _sc_scatter_body(vals_f_hbm, toks_hbm, dest_hbm, out_f_hbm, out_i_hbm,
                     val_v, tok_v, orow_f, orow_i, idx_v):
    wid = lax.axis_index("s") * _NC + lax.axis_index("c")
    b = lax.rem(wid, B)
    pltpu.sync_copy(vals_f_hbm.at[wid], val_v)
    pltpu.sync_copy(dest_hbm.at[b], idx_v)

    def body_f(i, carry):
        sl = pl.ds(i * 16, 16)
        plsc.store_scatter(orow_f, [idx_v[sl]], val_v[sl])
        return carry

    lax.fori_loop(0, FLAT // 16, body_f, 0)
    pltpu.sync_copy(orow_f, out_f_hbm.at[wid])

    @pl.when(wid < B)
    def _():
        pltpu.sync_copy(toks_hbm.at[wid], tok_v)

        def body_i(i, carry):
            sl = pl.ds(i * 16, 16)
            plsc.store_scatter(orow_i, [idx_v[sl]], tok_v[sl])
            return carry

        lax.fori_loop(0, FLAT // 16, body_i, 0)
        pltpu.sync_copy(orow_i, out_i_hbm.at[wid])


@functools.cache
def _make_sc_scatter():
    return pl.kernel(
        _sc_scatter_body,
        mesh=plsc.VectorSubcoreMesh(
            core_axis_name="c", subcore_axis_name="s",
            num_cores=_NC, num_subcores=_NS),
        out_type=[
            jax.ShapeDtypeStruct((2 * B, FLAT), jnp.float32),
            jax.ShapeDtypeStruct((B, FLAT), jnp.int32),
        ],
        scratch_types=[
            pltpu.VMEM((FLAT,), jnp.float32),
            pltpu.VMEM((FLAT,), jnp.int32),
            pltpu.VMEM((FLAT,), jnp.float32),
            pltpu.VMEM((FLAT,), jnp.int32),
            pltpu.VMEM((FLAT,), jnp.int32),
        ],
        compiler_params=pltpu.CompilerParams(needs_layout_passes=False),
    )


@jax.jit
def kernel(input, W_in, W_h, W_out):
    xt = input.reshape(R, D_IN).T
    toks_r, lp_r, ent_r, dest_r = _rnn_call(xt, W_in.T, W_h.T, W_out.T)
    vals_f = jnp.concatenate(
        [lp_r.reshape(B, FLAT), ent_r.reshape(B, FLAT)], axis=0)
    out_f, out_i = _make_sc_scatter()(vals_f, toks_r.reshape(B, FLAT),
                                      dest_r.reshape(B, FLAT))
    return (out_i, out_f[:B], out_f[B:])


# speedup vs baseline: 1.1277x; 1.0051x over previous
import functools

import jax
import jax.numpy as jnp
from jax import lax
from jax.experimental import pallas as pl
from jax.experimental.pallas import tpu as pltpu
from jax.experimental.pallas import tpu_sc as plsc

B, S, D_IN, HID, VOCAB, MAX_LEN = 16, 8, 256, 512, 128, 512
R = B * S
FLAT = S * MAX_LEN


def _rnn_body(xt_ref, wint_ref, wht_ref, woutt_ref,
              tok_out_ref, lp_out_ref, ent_out_ref, dest_out_ref,
              tok_ref, lp_ref, ent_ref):
    h0 = jnp.tanh(jnp.dot(wint_ref[...], xt_ref[...],
                          preferred_element_type=jnp.float32))
    wht = wht_ref[...]
    woutt = woutt_ref[...]
    vocab_iota = lax.broadcasted_iota(jnp.int32, (VOCAB, R), 0)

    def step(t, h):
        h = jnp.tanh(jnp.dot(wht, h, preferred_element_type=jnp.float32))
        logits = jnp.dot(woutt, h, preferred_element_type=jnp.float32)
        m = jnp.max(logits, axis=0, keepdims=True)
        shifted = logits - m
        e = jnp.exp(shifted)
        ssum = jnp.sum(e, axis=0, keepdims=True)
        tok = jnp.min(jnp.where(logits == m, vocab_iota, VOCAB),
                      axis=0, keepdims=True)
        logs = jnp.log(ssum)
        lp = -logs
        ent = logs - jnp.sum(e * shifted, axis=0, keepdims=True) / ssum
        tok_ref[pl.ds(t, 1), :] = tok
        lp_ref[pl.ds(t, 1), :] = lp
        ent_ref[pl.ds(t, 1), :] = ent
        return h

    lax.fori_loop(0, MAX_LEN, step, h0, unroll=32)

    toks = tok_ref[...]
    pos = lax.broadcasted_iota(jnp.int32, (MAX_LEN, R), 0)
    fz = jnp.min(jnp.where(toks == 0, pos, MAX_LEN), axis=0, keepdims=True)
    length = jnp.minimum(fz + 1, MAX_LEN)
    p = length - 1
    p_f = p.astype(jnp.float32)

    r0 = lax.broadcasted_iota(jnp.int32, (R, R), 0)
    r1 = lax.broadcasted_iota(jnp.int32, (R, R), 1)
    same_b = (r0 // S) == (r1 // S)
    mt_strict = jnp.where(same_b & (r0 < r1), 1.0, 0.0).astype(jnp.float32)
    mt_block = jnp.where(same_b, 1.0, 0.0).astype(jnp.float32)
    fo = jnp.dot(p_f, mt_strict, preferred_element_type=jnp.float32,
                 precision=lax.Precision.HIGHEST)
    tf = jnp.dot(p_f, mt_block, preferred_element_type=jnp.float32,
                 precision=lax.Precision.HIGHEST)
    s_idx = lax.broadcasted_iota(jnp.int32, (1, R), 1) % S
    lo = s_idx.astype(jnp.float32) * MAX_LEN - fo
    pos_f = pos.astype(jnp.float32)
    dest_f = jnp.where(pos < p, fo + pos_f, tf + lo + pos_f - p_f)

    tok_out_ref[...] = toks.T
    lp_out_ref[...] = lp_ref[...].T
    ent_out_ref[...] = ent_ref[...].T
    dest_out_ref[...] = dest_f.astype(jnp.int32).T


_rnn_call = pl.pallas_call(
    _rnn_body,
    out_shape=[
        jax.ShapeDtypeStruct((R, MAX_LEN), jnp.int32),
        jax.ShapeDtypeStruct((R, MAX_LEN), jnp.float32),
        jax.ShapeDtypeStruct((R, MAX_LEN), jnp.float32),
        jax.ShapeDtypeStruct((R, MAX_LEN), jnp.int32),
    ],
    scratch_shapes=[
        pltpu.VMEM((MAX_LEN, R), jnp.int32),
        pltpu.VMEM((MAX_LEN, R), jnp.float32),
        pltpu.VMEM((MAX_LEN, R), jnp.float32),
    ],
)


_NC, _NS = 2, 16
_NW = _NC * _NS


def _sc_scatter_body(vals_f_hbm, toks_hbm, dest_hbm, out_f_hbm, out_i_hbm,
                     val_v, tok_v, orow_f, orow_i, idx_v):
    wid = lax.axis_index("s") * _NC + lax.axis_index("c")
    b = lax.rem(wid, B)
    pltpu.sync_copy(vals_f_hbm.at[wid], val_v)
    pltpu.sync_copy(dest_hbm.at[b], idx_v)

    def body_f(i, carry):
        sl = pl.ds(i * 16, 16)
        plsc.store_scatter(orow_f, [idx_v[sl]], val_v[sl])
        return carry

    lax.fori_loop(0, FLAT // 16, body_f, 0)
    pltpu.sync_copy(orow_f, out_f_hbm.at[wid])

    @pl.when(wid < B)
    def _():
        pltpu.sync_copy(toks_hbm.at[wid], tok_v)

        def body_i(i, carry):
            sl = pl.ds(i * 16, 16)
            plsc.store_scatter(orow_i, [idx_v[sl]], tok_v[sl])
            return carry

        lax.fori_loop(0, FLAT // 16, body_i, 0)
        pltpu.sync_copy(orow_i, out_i_hbm.at[wid])


@functools.cache
def _make_sc_scatter():
    return pl.kernel(
        _sc_scatter_body,
        mesh=plsc.VectorSubcoreMesh(
            core_axis_name="c", subcore_axis_name="s",
            num_cores=_NC, num_subcores=_NS),
        out_type=[
            jax.ShapeDtypeStruct((2 * B, FLAT), jnp.float32),
            jax.ShapeDtypeStruct((B, FLAT), jnp.int32),
        ],
        scratch_types=[
            pltpu.VMEM((FLAT,), jnp.float32),
            pltpu.VMEM((FLAT,), jnp.int32),
            pltpu.VMEM((FLAT,), jnp.float32),
            pltpu.VMEM((FLAT,), jnp.int32),
            pltpu.VMEM((FLAT,), jnp.int32),
        ],
        compiler_params=pltpu.CompilerParams(needs_layout_passes=False),
    )


@jax.jit
def kernel(input, W_in, W_h, W_out):
    xt = input.reshape(R, D_IN).T
    toks_r, lp_r, ent_r, dest_r = _rnn_call(xt, W_in.T, W_h.T, W_out.T)
    vals_f = jnp.concatenate(
        [lp_r.reshape(B, FLAT), ent_r.reshape(B, FLAT)], axis=0)
    out_f, out_i = _make_sc_scatter()(vals_f, toks_r.reshape(B, FLAT),
                                      dest_r.reshape(B, FLAT))
    return (out_i, out_f[:B], out_f[B:])


# speedup vs baseline: 1.1676x; 1.0354x over previous
import functools

import jax
import jax.numpy as jnp
from jax import lax
from jax.experimental import pallas as pl
from jax.experimental.pallas import tpu as pltpu
from jax.experimental.pallas import tpu_sc as plsc

B, S, D_IN, HID, VOCAB, MAX_LEN = 16, 8, 256, 512, 128, 512
R = B * S
FLAT = S * MAX_LEN


def _tdot(w, x):
    return lax.dot_general(w, x, (((0,), (0,)), ((), ())),
                           preferred_element_type=jnp.float32)


def _rnn_body(x_ref, win_ref, wh_ref, wout_ref,
              tok_out_ref, lp_out_ref, ent_out_ref, dest_out_ref,
              tok_ref, lp_ref, ent_ref):
    h0 = jnp.tanh(lax.dot_general(win_ref[...], x_ref[...],
                                  (((0,), (1,)), ((), ())),
                                  preferred_element_type=jnp.float32))
    wh = wh_ref[...]
    wout = wout_ref[...]
    vocab_iota = lax.broadcasted_iota(jnp.int32, (VOCAB, R), 0)

    def step(t, h):
        h = jnp.tanh(_tdot(wh, h))
        logits = _tdot(wout, h)
        m = jnp.max(logits, axis=0, keepdims=True)
        shifted = logits - m
        e = jnp.exp(shifted)
        ssum = jnp.sum(e, axis=0, keepdims=True)
        tok = jnp.min(jnp.where(logits == m, vocab_iota, VOCAB),
                      axis=0, keepdims=True)
        logs = jnp.log(ssum)
        lp = -logs
        ent = logs - jnp.sum(e * shifted, axis=0, keepdims=True) / ssum
        tok_ref[pl.ds(t, 1), :] = tok
        lp_ref[pl.ds(t, 1), :] = lp
        ent_ref[pl.ds(t, 1), :] = ent
        return h

    lax.fori_loop(0, MAX_LEN, step, h0, unroll=32)

    toks = tok_ref[...]
    pos = lax.broadcasted_iota(jnp.int32, (MAX_LEN, R), 0)
    fz = jnp.min(jnp.where(toks == 0, pos, MAX_LEN), axis=0, keepdims=True)
    length = jnp.minimum(fz + 1, MAX_LEN)
    p = length - 1
    p_f = p.astype(jnp.float32)

    r0 = lax.broadcasted_iota(jnp.int32, (R, R), 0)
    r1 = lax.broadcasted_iota(jnp.int32, (R, R), 1)
    same_b = (r0 // S) == (r1 // S)
    mt_strict = jnp.where(same_b & (r0 < r1), 1.0, 0.0).astype(jnp.float32)
    mt_block = jnp.where(same_b, 1.0, 0.0).astype(jnp.float32)
    fo = jnp.dot(p_f, mt_strict, preferred_element_type=jnp.float32,
                 precision=lax.Precision.HIGHEST)
    tf = jnp.dot(p_f, mt_block, preferred_element_type=jnp.float32,
                 precision=lax.Precision.HIGHEST)
    s_idx = lax.broadcasted_iota(jnp.int32, (1, R), 1) % S
    lo = s_idx.astype(jnp.float32) * MAX_LEN - fo
    pos_f = pos.astype(jnp.float32)
    dest_f = jnp.where(pos < p, fo + pos_f, tf + lo + pos_f - p_f)

    tok_out_ref[...] = toks.T
    lp_out_ref[...] = lp_ref[...].T
    ent_out_ref[...] = ent_ref[...].T
    dest_out_ref[...] = dest_f.astype(jnp.int32).T


_rnn_call = pl.pallas_call(
    _rnn_body,
    out_shape=[
        jax.ShapeDtypeStruct((R, MAX_LEN), jnp.int32),
        jax.ShapeDtypeStruct((R, MAX_LEN), jnp.float32),
        jax.ShapeDtypeStruct((R, MAX_LEN), jnp.float32),
        jax.ShapeDtypeStruct((R, MAX_LEN), jnp.int32),
    ],
    scratch_shapes=[
        pltpu.VMEM((MAX_LEN, R), jnp.int32),
        pltpu.VMEM((MAX_LEN, R), jnp.float32),
        pltpu.VMEM((MAX_LEN, R), jnp.float32),
    ],
)


_NC, _NS = 2, 16
_NW = _NC * _NS


def _sc_scatter_body(toks_hbm, lp_hbm, ent_hbm, dest_hbm,
                     out_m_hbm, out_lp_hbm, out_ent_hbm,
                     val_v, tok_v, orow_f, orow_i, idx_v):
    wid = lax.axis_index("s") * _NC + lax.axis_index("c")
    b = lax.rem(wid, B)
    pltpu.sync_copy(dest_hbm.at[b], idx_v)

    @pl.when(wid < B)
    def _():
        pltpu.sync_copy(lp_hbm.at[b], val_v)

    @pl.when(wid >= B)
    def _():
        pltpu.sync_copy(ent_hbm.at[b], val_v)

    def body_f(i, carry):
        sl = pl.ds(i * 16, 16)
        plsc.store_scatter(orow_f, [idx_v[sl]], val_v[sl])
        return carry

    lax.fori_loop(0, FLAT // 16, body_f, 0)

    @pl.when(wid < B)
    def _():
        pltpu.sync_copy(orow_f, out_lp_hbm.at[b])
        pltpu.sync_copy(toks_hbm.at[b], tok_v)

        def body_i(i, carry):
            sl = pl.ds(i * 16, 16)
            plsc.store_scatter(orow_i, [idx_v[sl]], tok_v[sl])
            return carry

        lax.fori_loop(0, FLAT // 16, body_i, 0)
        pltpu.sync_copy(orow_i, out_m_hbm.at[b])

    @pl.when(wid >= B)
    def _():
        pltpu.sync_copy(orow_f, out_ent_hbm.at[b])


@functools.cache
def _make_sc_scatter():
    return pl.kernel(
        _sc_scatter_body,
        mesh=plsc.VectorSubcoreMesh(
            core_axis_name="c", subcore_axis_name="s",
            num_cores=_NC, num_subcores=_NS),
        out_type=[
            jax.ShapeDtypeStruct((B, FLAT), jnp.int32),
            jax.ShapeDtypeStruct((B, FLAT), jnp.float32),
            jax.ShapeDtypeStruct((B, FLAT), jnp.float32),
        ],
        scratch_types=[
            pltpu.VMEM((FLAT,), jnp.float32),
            pltpu.VMEM((FLAT,), jnp.int32),
            pltpu.VMEM((FLAT,), jnp.float32),
            pltpu.VMEM((FLAT,), jnp.int32),
            pltpu.VMEM((FLAT,), jnp.int32),
        ],
        compiler_params=pltpu.CompilerParams(needs_layout_passes=False),
    )


@jax.jit
def kernel(input, W_in, W_h, W_out):
    x = input.reshape(R, D_IN)
    toks_r, lp_r, ent_r, dest_r = _rnn_call(x, W_in, W_h, W_out)
    return _make_sc_scatter()(toks_r.reshape(B, FLAT),
                              lp_r.reshape(B, FLAT),
                              ent_r.reshape(B, FLAT),
                              dest_r.reshape(B, FLAT))


# speedup vs baseline: 1.1749x; 1.0062x over previous
import functools

import jax
import jax.numpy as jnp
from jax import lax
from jax.experimental import pallas as pl
from jax.experimental.pallas import tpu as pltpu
from jax.experimental.pallas import tpu_sc as plsc

B, S, D_IN, HID, VOCAB, MAX_LEN = 16, 8, 256, 512, 128, 512
R = B * S
FLAT = S * MAX_LEN


def _tdot(w, x):
    return lax.dot_general(w, x, (((0,), (0,)), ((), ())),
                           preferred_element_type=jnp.float32)


def _rnn_body(x_ref, win_ref, wh_ref, wout_ref,
              tok_out_ref, lp_out_ref, ent_out_ref, dest_out_ref,
              tok_ref, lp_ref, ent_ref):
    h0 = jnp.tanh(lax.dot_general(win_ref[...], x_ref[...],
                                  (((0,), (1,)), ((), ())),
                                  preferred_element_type=jnp.float32))
    wh = wh_ref[...]
    wout = wout_ref[...]
    vocab_iota = lax.broadcasted_iota(jnp.int32, (VOCAB, R), 0)

    def step(t, h):
        h = jnp.tanh(_tdot(wh, h))
        logits = _tdot(wout, h)
        m = jnp.max(logits, axis=0, keepdims=True)
        shifted = logits - m
        e = jnp.exp(shifted)
        ssum = jnp.sum(e, axis=0, keepdims=True)
        tok = jnp.min(jnp.where(logits == m, vocab_iota, VOCAB),
                      axis=0, keepdims=True)
        logs = jnp.log(ssum)
        lp = -logs
        ent = logs - jnp.sum(e * shifted, axis=0, keepdims=True) / ssum
        tok_ref[pl.ds(t, 1), :] = tok
        lp_ref[pl.ds(t, 1), :] = lp
        ent_ref[pl.ds(t, 1), :] = ent
        return h

    lax.fori_loop(0, MAX_LEN, step, h0, unroll=32)

    toks = tok_ref[...]
    pos = lax.broadcasted_iota(jnp.int32, (MAX_LEN, R), 0)
    fz = jnp.min(jnp.where(toks == 0, pos, MAX_LEN), axis=0, keepdims=True)
    length = jnp.minimum(fz + 1, MAX_LEN)
    p = length - 1
    p_f = p.astype(jnp.float32)

    r0 = lax.broadcasted_iota(jnp.int32, (R, R), 0)
    r1 = lax.broadcasted_iota(jnp.int32, (R, R), 1)
    same_b = (r0 // S) == (r1 // S)
    mt_strict = jnp.where(same_b & (r0 < r1), 1.0, 0.0).astype(jnp.float32)
    mt_block = jnp.where(same_b, 1.0, 0.0).astype(jnp.float32)
    fo = jnp.dot(p_f, mt_strict, preferred_element_type=jnp.float32,
                 precision=lax.Precision.HIGHEST)
    tf = jnp.dot(p_f, mt_block, preferred_element_type=jnp.float32,
                 precision=lax.Precision.HIGHEST)
    s_idx = lax.broadcasted_iota(jnp.int32, (1, R), 1) % S
    lo = s_idx.astype(jnp.float32) * MAX_LEN - fo
    pos_f = pos.astype(jnp.float32)
    dest_f = jnp.where(pos < p, fo + pos_f, tf + lo + pos_f - p_f)

    tok_out_ref[...] = toks.T
    lp_out_ref[...] = lp_ref[...].T
    ent_out_ref[...] = ent_ref[...].T
    dest_out_ref[...] = dest_f.astype(jnp.int32).T


_rnn_call = pl.pallas_call(
    _rnn_body,
    out_shape=[
        jax.ShapeDtypeStruct((R, MAX_LEN), jnp.int32),
        jax.ShapeDtypeStruct((R, MAX_LEN), jnp.float32),
        jax.ShapeDtypeStruct((R, MAX_LEN), jnp.float32),
        jax.ShapeDtypeStruct((R, MAX_LEN), jnp.int32),
    ],
    scratch_shapes=[
        pltpu.VMEM((MAX_LEN, R), jnp.int32),
        pltpu.VMEM((MAX_LEN, R), jnp.float32),
        pltpu.VMEM((MAX_LEN, R), jnp.float32),
    ],
)


_NC, _NS = 2, 16
_NW = _NC * _NS


def _sc_scatter_body(toks_hbm, lp_hbm, ent_hbm, dest_hbm,
                     out_m_hbm, out_lp_hbm, out_ent_hbm,
                     val_v, tok_v, orow_f, orow_i, idx_v):
    wid = lax.axis_index("s") * _NC + lax.axis_index("c")
    b = lax.rem(wid, B)
    pltpu.sync_copy(dest_hbm.at[b], idx_v)

    @pl.when(wid < B)
    def _():
        pltpu.sync_copy(lp_hbm.at[b], val_v)

    @pl.when(wid >= B)
    def _():
        pltpu.sync_copy(ent_hbm.at[b], val_v)

    def body_f(i, carry):
        sl = pl.ds(i * 16, 16)
        plsc.store_scatter(orow_f, [idx_v[sl]], val_v[sl])
        return carry

    lax.fori_loop(0, FLAT // 16, body_f, 0)

    @pl.when(wid < B)
    def _():
        pltpu.sync_copy(orow_f, out_lp_hbm.at[b])
        pltpu.sync_copy(toks_hbm.at[b], tok_v)

        def body_i(i, carry):
            sl = pl.ds(i * 16, 16)
            plsc.store_scatter(orow_i, [idx_v[sl]], tok_v[sl])
            return carry

        lax.fori_loop(0, FLAT // 16, body_i, 0)
        pltpu.sync_copy(orow_i, out_m_hbm.at[b])

    @pl.when(wid >= B)
    def _():
        pltpu.sync_copy(orow_f, out_ent_hbm.at[b])


@functools.cache
def _make_sc_scatter():
    return pl.kernel(
        _sc_scatter_body,
        mesh=plsc.VectorSubcoreMesh(
            core_axis_name="c", subcore_axis_name="s",
            num_cores=_NC, num_subcores=_NS),
        out_type=[
            jax.ShapeDtypeStruct((B, FLAT), jnp.int32),
            jax.ShapeDtypeStruct((B, FLAT), jnp.float32),
            jax.ShapeDtypeStruct((B, FLAT), jnp.float32),
        ],
        scratch_types=[
            pltpu.VMEM((FLAT,), jnp.float32),
            pltpu.VMEM((FLAT,), jnp.int32),
            pltpu.VMEM((FLAT,), jnp.float32),
            pltpu.VMEM((FLAT,), jnp.int32),
            pltpu.VMEM((FLAT,), jnp.int32),
        ],
        compiler_params=pltpu.CompilerParams(needs_layout_passes=False),
    )


@jax.jit
def kernel(input, W_in, W_h, W_out):
    x = input.reshape(R, D_IN)
    toks_r, lp_r, ent_r, dest_r = _rnn_call(x, W_in, W_h, W_out)
    return tuple(_make_sc_scatter()(toks_r.reshape(B, FLAT),
                                    lp_r.reshape(B, FLAT),
                                    ent_r.reshape(B, FLAT),
                                    dest_r.reshape(B, FLAT)))
